# Initial kernel scaffold; baseline (speedup 1.0000x reference)
import jax
import jax.numpy as jnp
from jax.experimental import pallas as pl


def kernel(set_inputs, mag, mask):
    raise NotImplementedError("write your pallas kernel here")


# speedup vs baseline: 1.0322x; 1.0322x over previous
import jax
import jax.numpy as jnp
from jax import lax
from jax.experimental import pallas as pl
from jax.experimental.pallas import tpu as pltpu
from jax.experimental.pallas import tpu_sc as plsc

B, N, D = 16, 4096, 256
NC, NS = 2, 16
BPC = B // NC
NV = N // 16
HALF = N // 2
CHUNK = 128
NCHUNK = HALF // CHUNK
INF_BITS = 0x7F800000


def _body(x_hbm, mag_hbm, mask_hbm, out_hbm,
          fbuf, mbuf, kb0, ib0, kb1, ib1, hist, obuf, rbuf,
          cidx, rows, perm_sh, sem):
    c = lax.axis_index("c")
    s = lax.axis_index("s")
    lane = lax.iota(jnp.int32, 16)

    @pl.when(s < BPC)
    def _sort():
        b = c * BPC + s
        pltpu.sync_copy(mag_hbm.at[b], fbuf)
        pltpu.sync_copy(mask_hbm.at[b], mbuf)

        def init_body(t, carry):
            ki = jnp.bitwise_and(fbuf[pl.ds(t * 16, 16)], 0x7FFFFFFF)
            m = mbuf[pl.ds(t * 16, 16)]
            kb0[pl.ds(t * 16, 16)] = jnp.where(m != 0, ki, INF_BITS)
            ib0[pl.ds(t * 16, 16)] = lane + t * 16
            return carry
        lax.fori_loop(0, NV, init_body, jnp.int32(0))

        bufs = [(kb0, ib0), (kb1, ib1)]
        ones = jnp.ones((16,), jnp.int32)
        zeros = jnp.zeros((16,), jnp.int32)
        for p in range(4):
            kin, iin = bufs[p % 2]
            kout, iout = bufs[(p + 1) % 2]
            shift = 8 * p

            def zero_body(t, carry):
                hist[pl.ds(t * 16, 16)] = zeros
                rbuf[pl.ds(t * 16, 16)] = zeros
                return carry
            lax.fori_loop(0, NV, zero_body, jnp.int32(0))

            def hist_body(t, carry):
                gidx = lane * NV + t
                k = plsc.load_gather(kin, [gidx])
                digit = jnp.bitwise_and(jnp.right_shift(k, shift), 255)
                plsc.addupdate_scatter(hist, [digit * 16 + lane], ones)
                return carry
            lax.fori_loop(0, NV, hist_body, jnp.int32(0))

            def scan_body(dd, base):
                h = hist[pl.ds(dd * 16, 16)]
                inc = jnp.cumsum(h)
                obuf[pl.ds(dd * 16, 16)] = base + inc - h
                return base + jnp.sum(h)
            lax.fori_loop(0, 256, scan_body, jnp.int32(0))

            def perm_body(t, carry):
                gidx = lane * NV + t
                k = plsc.load_gather(kin, [gidx])
                v = plsc.load_gather(iin, [gidx])
                digit = jnp.bitwise_and(jnp.right_shift(k, shift), 255)
                hidx = digit * 16 + lane
                o = plsc.load_gather(obuf, [hidx])
                r = plsc.load_gather(rbuf, [hidx])
                pos = o + r
                plsc.store_scatter(rbuf, [hidx], r + 1)
                plsc.store_scatter(kout, [pos], k)
                plsc.store_scatter(iout, [pos], v)
                return carry
            lax.fori_loop(0, NV, perm_body, jnp.int32(0))

        base_row = b * N

        def add_body(t, carry):
            ib0[pl.ds(t * 16, 16)] = ib0[pl.ds(t * 16, 16)] + base_row
            return carry
        lax.fori_loop(0, NV, add_body, jnp.int32(0))
        pltpu.sync_copy(ib0, perm_sh.at[s])

    plsc.subcore_barrier()

    j = s // 2
    b = c * BPC + j
    row0 = (s % 2) * HALF

    def g_body(ck, carry):
        start = row0 + ck * CHUNK
        pltpu.sync_copy(perm_sh.at[j, pl.ds(start, CHUNK)], cidx)
        pltpu.async_copy(x_hbm.at[cidx], rows, sem).wait()
        pltpu.sync_copy(rows, out_hbm.at[pl.ds(b * N + start, CHUNK)])
        return carry
    lax.fori_loop(0, NCHUNK, g_body, jnp.int32(0))


_sorter = pl.kernel(
    _body,
    out_type=jax.ShapeDtypeStruct((B * N, D), jnp.float32),
    mesh=plsc.VectorSubcoreMesh(core_axis_name="c", subcore_axis_name="s"),
    compiler_params=pltpu.CompilerParams(needs_layout_passes=False),
    scratch_types=[
        pltpu.VMEM((N,), jnp.int32),
        pltpu.VMEM((N,), jnp.int32),
        pltpu.VMEM((N,), jnp.int32),
        pltpu.VMEM((N,), jnp.int32),
        pltpu.VMEM((N,), jnp.int32),
        pltpu.VMEM((N,), jnp.int32),
        pltpu.VMEM((4096,), jnp.int32),
        pltpu.VMEM((4096,), jnp.int32),
        pltpu.VMEM((4096,), jnp.int32),
        pltpu.VMEM((CHUNK,), jnp.int32),
        pltpu.VMEM((CHUNK, D), jnp.float32),
        pltpu.VMEM_SHARED((BPC, N), jnp.int32),
        pltpu.SemaphoreType.DMA,
    ],
)


@jax.jit
def kernel(set_inputs, mag, mask):
    x = set_inputs.reshape(B * N, D)
    mag2 = lax.bitcast_convert_type(mag.reshape(B, N), jnp.int32)
    mask2 = mask.reshape(B, N).astype(jnp.int32)
    out = _sorter(x, mag2, mask2)
    return out.reshape(B, N, D)


# speedup vs baseline: 1.1084x; 1.0738x over previous
import jax
import jax.numpy as jnp
from jax import lax
from jax.experimental import pallas as pl
from jax.experimental.pallas import tpu as pltpu
from jax.experimental.pallas import tpu_sc as plsc

B, N, D = 16, 4096, 256
NC, NS = 2, 16
BPC = B // NC
NV = N // 16
HALF = N // 2
CHUNK = 128
NPAIR = HALF // (2 * CHUNK)
INF_BITS = 0x7F800000


def _body(x_hbm, mag_hbm, mask_hbm, out_hbm,
          fbuf, mbuf, kb0, ib0, kb1, ib1, hist, obuf, rbuf,
          idxbuf, rows0, rows1, perm_sh, sem0, sem1):
    c = lax.axis_index("c")
    s = lax.axis_index("s")
    lane = lax.iota(jnp.int32, 16)

    @pl.when(s < BPC)
    def _sort():
        b = c * BPC + s
        pltpu.sync_copy(mag_hbm.at[b], fbuf)
        pltpu.sync_copy(mask_hbm.at[b], mbuf)

        def init_body(t, carry):
            ki = jnp.bitwise_and(fbuf[pl.ds(t * 16, 16)], 0x7FFFFFFF)
            m = mbuf[pl.ds(t * 16, 16)]
            kb0[pl.ds(t * 16, 16)] = jnp.where(m != 0, ki, INF_BITS)
            ib0[pl.ds(t * 16, 16)] = lane + t * 16
            hist[pl.ds(t * 16, 16)] = jnp.zeros((16,), jnp.int32)
            rbuf[pl.ds(t * 16, 16)] = jnp.zeros((16,), jnp.int32)
            return carry
        lax.fori_loop(0, NV, init_body, jnp.int32(0), unroll=4)

        bufs = [(kb0, ib0), (kb1, ib1)]
        ones = jnp.ones((16,), jnp.int32)
        zeros = jnp.zeros((16,), jnp.int32)
        for p in range(4):
            kin, iin = bufs[p % 2]
            kout, iout = bufs[(p + 1) % 2]
            shift = 8 * p

            def hist_body(t, carry):
                gidx = lane * NV + t
                k = plsc.load_gather(kin, [gidx])
                digit = jnp.bitwise_and(jnp.right_shift(k, shift), 255)
                plsc.addupdate_scatter(hist, [digit * 16 + lane], ones)
                return carry
            lax.fori_loop(0, NV, hist_body, jnp.int32(0), unroll=4)

            def scan_body(dd, base):
                h = hist[pl.ds(dd * 16, 16)]
                inc = jnp.cumsum(h)
                obuf[pl.ds(dd * 16, 16)] = base + inc - h
                hist[pl.ds(dd * 16, 16)] = zeros
                return base + jnp.sum(h)
            lax.fori_loop(0, 256, scan_body, jnp.int32(0), unroll=2)

            def perm_body(t, carry):
                gidx = lane * NV + t
                k = plsc.load_gather(kin, [gidx])
                v = plsc.load_gather(iin, [gidx])
                digit = jnp.bitwise_and(jnp.right_shift(k, shift), 255)
                hidx = digit * 16 + lane
                o = plsc.load_gather(obuf, [hidx])
                r = plsc.load_gather(rbuf, [hidx])
                pos = o + r
                plsc.store_scatter(rbuf, [hidx], r + 1)
                plsc.store_scatter(kout, [pos], k)
                plsc.store_scatter(iout, [pos], v)
                return carry
            lax.fori_loop(0, NV, perm_body, jnp.int32(0), unroll=2)

            if p < 3:
                def rezero_body(t, carry):
                    rbuf[pl.ds(t * 16, 16)] = zeros
                    return carry
                lax.fori_loop(0, 256, rezero_body, jnp.int32(0), unroll=4)

        base_row = b * N

        def add_body(t, carry):
            ib0[pl.ds(t * 16, 16)] = ib0[pl.ds(t * 16, 16)] + base_row
            return carry
        lax.fori_loop(0, NV, add_body, jnp.int32(0), unroll=4)
        pltpu.sync_copy(ib0, perm_sh.at[s])

    plsc.subcore_barrier()

    j = s // 2
    b = c * BPC + j
    out_base = b * N + (s % 2) * HALF
    pltpu.sync_copy(perm_sh.at[j, pl.ds((s % 2) * HALF, HALF)], idxbuf)

    def fire(ck, rbuf_, sem_):
        return pltpu.async_copy(
            x_hbm.at[idxbuf.at[pl.ds(ck * CHUNK, CHUNK)]], rbuf_, sem_)

    fire(0, rows0, sem0)

    def g_body(i, carry):
        c0 = 2 * i
        fire(c0 + 1, rows1, sem1)
        pltpu.make_async_copy(
            x_hbm.at[idxbuf.at[pl.ds(c0 * CHUNK, CHUNK)]], rows0, sem0).wait()
        pltpu.sync_copy(rows0, out_hbm.at[pl.ds(out_base + c0 * CHUNK, CHUNK)])

        @pl.when(i < NPAIR - 1)
        def _():
            fire(c0 + 2, rows0, sem0)
        pltpu.make_async_copy(
            x_hbm.at[idxbuf.at[pl.ds((c0 + 1) * CHUNK, CHUNK)]], rows1,
            sem1).wait()
        pltpu.sync_copy(rows1,
                        out_hbm.at[pl.ds(out_base + (c0 + 1) * CHUNK, CHUNK)])
        return carry
    lax.fori_loop(0, NPAIR, g_body, jnp.int32(0))


_sorter = pl.kernel(
    _body,
    out_type=jax.ShapeDtypeStruct((B * N, D), jnp.float32),
    mesh=plsc.VectorSubcoreMesh(core_axis_name="c", subcore_axis_name="s"),
    compiler_params=pltpu.CompilerParams(needs_layout_passes=False),
    scratch_types=[
        pltpu.VMEM((N,), jnp.int32),
        pltpu.VMEM((N,), jnp.int32),
        pltpu.VMEM((N,), jnp.int32),
        pltpu.VMEM((N,), jnp.int32),
        pltpu.VMEM((N,), jnp.int32),
        pltpu.VMEM((N,), jnp.int32),
        pltpu.VMEM((4096,), jnp.int32),
        pltpu.VMEM((4096,), jnp.int32),
        pltpu.VMEM((4096,), jnp.int32),
        pltpu.VMEM((HALF,), jnp.int32),
        pltpu.VMEM((CHUNK, D), jnp.float32),
        pltpu.VMEM((CHUNK, D), jnp.float32),
        pltpu.VMEM_SHARED((BPC, N), jnp.int32),
        pltpu.SemaphoreType.DMA,
        pltpu.SemaphoreType.DMA,
    ],
)


@jax.jit
def kernel(set_inputs, mag, mask):
    x = set_inputs.reshape(B * N, D)
    mag2 = lax.bitcast_convert_type(mag.reshape(B, N), jnp.int32)
    mask2 = mask.reshape(B, N).astype(jnp.int32)
    out = _sorter(x, mag2, mask2)
    return out.reshape(B, N, D)


# speedup vs baseline: 1.1154x; 1.0063x over previous
import jax
import jax.numpy as jnp
from jax import lax
from jax.experimental import pallas as pl
from jax.experimental.pallas import tpu as pltpu
from jax.experimental.pallas import tpu_sc as plsc

B, N, D = 16, 4096, 256
NC, NS = 2, 16
BPC = B // NC
NV = N // 16
HALF = N // 2
CHUNK = 64
NBUF = 4
NGRP = HALF // (NBUF * CHUNK)
INF_BITS = 0x7F800000


def _body(x_hbm, mag_hbm, mask_hbm, out_hbm,
          fbuf, mbuf, kb0, ib0, kb1, ib1, hist, obuf, tot,
          idxbuf, rows0, rows1, rows2, rows3, perm_sh,
          sem0, sem1, sem2, sem3):
    c = lax.axis_index("c")
    s = lax.axis_index("s")
    lane = lax.iota(jnp.int32, 16)
    rows = [rows0, rows1, rows2, rows3]
    sems = [sem0, sem1, sem2, sem3]

    @pl.when(s < BPC)
    def _sort():
        b = c * BPC + s
        pltpu.sync_copy(mag_hbm.at[b], fbuf)
        pltpu.sync_copy(mask_hbm.at[b], mbuf)

        zeros = jnp.zeros((16,), jnp.int32)
        ones = jnp.ones((16,), jnp.int32)

        def init_body(t, carry):
            ki = jnp.bitwise_and(fbuf[pl.ds(t * 16, 16)], 0x7FFFFFFF)
            m = mbuf[pl.ds(t * 16, 16)]
            kb0[pl.ds(t * 16, 16)] = jnp.where(m != 0, ki, INF_BITS)
            ib0[pl.ds(t * 16, 16)] = lane + t * 16
            hist[pl.ds(t * 16, 16)] = zeros
            return carry
        lax.fori_loop(0, NV, init_body, jnp.int32(0), unroll=4)

        bufs = [(kb0, ib0), (kb1, ib1)]
        for p in range(4):
            kin, iin = bufs[p % 2]
            kout, iout = bufs[(p + 1) % 2]
            shift = 8 * p

            def hist_body(t, carry):
                gidx = lane * NV + t
                k = plsc.load_gather(kin, [gidx])
                digit = jnp.bitwise_and(jnp.right_shift(k, shift), 255)
                plsc.addupdate_scatter(hist, [digit * 16 + lane], ones)
                return carry
            lax.fori_loop(0, NV, hist_body, jnp.int32(0), unroll=4)

            def scan_a(dd, carry):
                h = hist[pl.ds(dd * 16, 16)]
                obuf[pl.ds(dd * 16, 16)] = jnp.cumsum(h) - h
                tot[dd] = jnp.sum(h)
                hist[pl.ds(dd * 16, 16)] = zeros
                return carry
            lax.fori_loop(0, 256, scan_a, jnp.int32(0), unroll=4)

            def scan_b(dd, carry):
                t = tot[dd]
                tot[dd] = carry
                return carry + t
            lax.fori_loop(0, 256, scan_b, jnp.int32(0), unroll=4)

            def scan_c(dd, carry):
                obuf[pl.ds(dd * 16, 16)] = obuf[pl.ds(dd * 16, 16)] + tot[dd]
                return carry
            lax.fori_loop(0, 256, scan_c, jnp.int32(0), unroll=4)

            def perm_body(t, carry):
                gidx = lane * NV + t
                k = plsc.load_gather(kin, [gidx])
                v = plsc.load_gather(iin, [gidx])
                digit = jnp.bitwise_and(jnp.right_shift(k, shift), 255)
                hidx = digit * 16 + lane
                pos = plsc.load_gather(obuf, [hidx])
                plsc.store_scatter(obuf, [hidx], pos + 1)
                plsc.store_scatter(kout, [pos], k)
                plsc.store_scatter(iout, [pos], v)
                return carry
            lax.fori_loop(0, NV, perm_body, jnp.int32(0), unroll=2)

        base_row = b * N

        def add_body(t, carry):
            ib0[pl.ds(t * 16, 16)] = ib0[pl.ds(t * 16, 16)] + base_row
            return carry
        lax.fori_loop(0, NV, add_body, jnp.int32(0), unroll=4)
        pltpu.sync_copy(ib0, perm_sh.at[s])

    plsc.subcore_barrier()

    j = s // 2
    b = c * BPC + j
    out_base = b * N + (s % 2) * HALF
    pltpu.sync_copy(perm_sh.at[j, pl.ds((s % 2) * HALF, HALF)], idxbuf)

    def fire(ck, rbuf_, sem_):
        pltpu.async_copy(
            x_hbm.at[idxbuf.at[pl.ds(ck * CHUNK, CHUNK)]], rbuf_, sem_)

    for bb in range(NBUF):
        fire(bb, rows[bb], sems[bb])

    def g_body(g, carry):
        for bb in range(NBUF):
            ck = g * NBUF + bb
            pltpu.make_async_copy(
                x_hbm.at[idxbuf.at[pl.ds(ck * CHUNK, CHUNK)]], rows[bb],
                sems[bb]).wait()
            pltpu.sync_copy(rows[bb],
                            out_hbm.at[pl.ds(out_base + ck * CHUNK, CHUNK)])

            @pl.when(g < NGRP - 1)
            def _():
                fire(ck + NBUF, rows[bb], sems[bb])
        return carry
    lax.fori_loop(0, NGRP, g_body, jnp.int32(0))


_sorter = pl.kernel(
    _body,
    out_type=jax.ShapeDtypeStruct((B * N, D), jnp.float32),
    mesh=plsc.VectorSubcoreMesh(core_axis_name="c", subcore_axis_name="s"),
    compiler_params=pltpu.CompilerParams(needs_layout_passes=False),
    scratch_types=[
        pltpu.VMEM((N,), jnp.int32),
        pltpu.VMEM((N,), jnp.int32),
        pltpu.VMEM((N,), jnp.int32),
        pltpu.VMEM((N,), jnp.int32),
        pltpu.VMEM((N,), jnp.int32),
        pltpu.VMEM((N,), jnp.int32),
        pltpu.VMEM((4096,), jnp.int32),
        pltpu.VMEM((4096,), jnp.int32),
        pltpu.SMEM((256,), jnp.int32),
        pltpu.VMEM((HALF,), jnp.int32),
        pltpu.VMEM((CHUNK, D), jnp.float32),
        pltpu.VMEM((CHUNK, D), jnp.float32),
        pltpu.VMEM((CHUNK, D), jnp.float32),
        pltpu.VMEM((CHUNK, D), jnp.float32),
        pltpu.VMEM_SHARED((BPC, N), jnp.int32),
        pltpu.SemaphoreType.DMA,
        pltpu.SemaphoreType.DMA,
        pltpu.SemaphoreType.DMA,
        pltpu.SemaphoreType.DMA,
    ],
)


@jax.jit
def kernel(set_inputs, mag, mask):
    x = set_inputs.reshape(B * N, D)
    mag2 = lax.bitcast_convert_type(mag.reshape(B, N), jnp.int32)
    mask2 = mask.reshape(B, N).astype(jnp.int32)
    out = _sorter(x, mag2, mask2)
    return out.reshape(B, N, D)


# speedup vs baseline: 1.3218x; 1.1851x over previous
import jax
import jax.numpy as jnp
from jax import lax
from jax.experimental import pallas as pl
from jax.experimental.pallas import tpu as pltpu
from jax.experimental.pallas import tpu_sc as plsc

B, N, D = 16, 4096, 256
NC, NS = 2, 16
BPC = B // NC
NV = N // 16
HALF = N // 2
CHUNK = 64
NBUF = 4
NGRP = HALF // (NBUF * CHUNK)
INF_BITS = 0x7F800000


def _body(x_hbm, mag_hbm, mask_hbm, out_hbm,
          fbuf, mbuf, kb0, ib0, kb1, ib1, hist, obuf, dbuf, rnk, tot,
          idxbuf, rows0, rows1, rows2, rows3, perm_sh,
          sem0, sem1, sem2, sem3):
    c = lax.axis_index("c")
    s = lax.axis_index("s")
    lane = lax.iota(jnp.int32, 16)
    rows = [rows0, rows1, rows2, rows3]
    sems = [sem0, sem1, sem2, sem3]

    @pl.when(s < BPC)
    def _sort():
        b = c * BPC + s
        pltpu.sync_copy(mag_hbm.at[b], fbuf)
        pltpu.sync_copy(mask_hbm.at[b], mbuf)

        zeros = jnp.zeros((16,), jnp.int32)
        ones = jnp.ones((16,), jnp.int32)

        @plsc.parallel_loop(0, NV, 1, unroll=4)
        def _init(t):
            ki = jnp.bitwise_and(fbuf[pl.ds(t * 16, 16)], 0x7FFFFFFF)
            m = mbuf[pl.ds(t * 16, 16)]
            kb0[pl.ds(t * 16, 16)] = jnp.where(m != 0, ki, INF_BITS)
            ib0[pl.ds(t * 16, 16)] = lane + t * 16
            hist[pl.ds(t * 16, 16)] = zeros

        bufs = [(kb0, ib0), (kb1, ib1)]
        for p in range(4):
            kin, iin = bufs[p % 2]
            kout, iout = bufs[(p + 1) % 2]
            shift = 8 * p

            @plsc.parallel_loop(0, NV, 1, unroll=4)
            def _dig(t):
                gidx = lane * NV + t
                k = plsc.load_gather(kin, [gidx])
                dbuf[pl.ds(t * 16, 16)] = jnp.bitwise_and(
                    jnp.right_shift(k, shift), 255)

            def hist_body(t, carry):
                hidx = dbuf[pl.ds(t * 16, 16)] * 16 + lane
                r = plsc.load_gather(hist, [hidx])
                rnk[pl.ds(t * 16, 16)] = r
                plsc.addupdate_scatter(hist, [hidx], ones)
                return carry
            lax.fori_loop(0, NV, hist_body, jnp.int32(0), unroll=4)

            @plsc.parallel_loop(0, 256, 1, unroll=4)
            def _scan_a(dd):
                h = hist[pl.ds(dd * 16, 16)]
                obuf[pl.ds(dd * 16, 16)] = jnp.cumsum(h) - h
                tot[dd] = jnp.sum(h)
                hist[pl.ds(dd * 16, 16)] = zeros

            def scan_b(dd, carry):
                t = tot[dd]
                tot[dd] = carry
                return carry + t
            lax.fori_loop(0, 256, scan_b, jnp.int32(0), unroll=4)

            @plsc.parallel_loop(0, 256, 1, unroll=4)
            def _scan_c(dd):
                obuf[pl.ds(dd * 16, 16)] = obuf[pl.ds(dd * 16, 16)] + tot[dd]

            @plsc.parallel_loop(0, NV, 1, unroll=4)
            def _perm(t):
                gidx = lane * NV + t
                k = plsc.load_gather(kin, [gidx])
                v = plsc.load_gather(iin, [gidx])
                hidx = dbuf[pl.ds(t * 16, 16)] * 16 + lane
                pos = plsc.load_gather(obuf, [hidx]) + rnk[pl.ds(t * 16, 16)]
                plsc.store_scatter(kout, [pos], k)
                plsc.store_scatter(iout, [pos], v)

        base_row = b * N

        @plsc.parallel_loop(0, NV, 1, unroll=4)
        def _add(t):
            ib0[pl.ds(t * 16, 16)] = ib0[pl.ds(t * 16, 16)] + base_row
        pltpu.sync_copy(ib0, perm_sh.at[s])

    plsc.subcore_barrier()

    j = s // 2
    b = c * BPC + j
    out_base = b * N + (s % 2) * HALF
    pltpu.sync_copy(perm_sh.at[j, pl.ds((s % 2) * HALF, HALF)], idxbuf)

    def fire(ck, rbuf_, sem_):
        pltpu.async_copy(
            x_hbm.at[idxbuf.at[pl.ds(ck * CHUNK, CHUNK)]], rbuf_, sem_)

    for bb in range(NBUF):
        fire(bb, rows[bb], sems[bb])

    def g_body(g, carry):
        for bb in range(NBUF):
            ck = g * NBUF + bb
            pltpu.make_async_copy(
                x_hbm.at[idxbuf.at[pl.ds(ck * CHUNK, CHUNK)]], rows[bb],
                sems[bb]).wait()
            pltpu.sync_copy(rows[bb],
                            out_hbm.at[pl.ds(out_base + ck * CHUNK, CHUNK)])

            @pl.when(g < NGRP - 1)
            def _():
                fire(ck + NBUF, rows[bb], sems[bb])
        return carry
    lax.fori_loop(0, NGRP, g_body, jnp.int32(0))


_sorter = pl.kernel(
    _body,
    out_type=jax.ShapeDtypeStruct((B * N, D), jnp.float32),
    mesh=plsc.VectorSubcoreMesh(core_axis_name="c", subcore_axis_name="s"),
    compiler_params=pltpu.CompilerParams(needs_layout_passes=False),
    scratch_types=[
        pltpu.VMEM((N,), jnp.int32),
        pltpu.VMEM((N,), jnp.int32),
        pltpu.VMEM((N,), jnp.int32),
        pltpu.VMEM((N,), jnp.int32),
        pltpu.VMEM((N,), jnp.int32),
        pltpu.VMEM((N,), jnp.int32),
        pltpu.VMEM((4096,), jnp.int32),
        pltpu.VMEM((4096,), jnp.int32),
        pltpu.VMEM((N,), jnp.int32),
        pltpu.VMEM((N,), jnp.int32),
        pltpu.SMEM((256,), jnp.int32),
        pltpu.VMEM((HALF,), jnp.int32),
        pltpu.VMEM((CHUNK, D), jnp.float32),
        pltpu.VMEM((CHUNK, D), jnp.float32),
        pltpu.VMEM((CHUNK, D), jnp.float32),
        pltpu.VMEM((CHUNK, D), jnp.float32),
        pltpu.VMEM_SHARED((BPC, N), jnp.int32),
        pltpu.SemaphoreType.DMA,
        pltpu.SemaphoreType.DMA,
        pltpu.SemaphoreType.DMA,
        pltpu.SemaphoreType.DMA,
    ],
)


@jax.jit
def kernel(set_inputs, mag, mask):
    x = set_inputs.reshape(B * N, D)
    mag2 = lax.bitcast_convert_type(mag.reshape(B, N), jnp.int32)
    mask2 = mask.reshape(B, N).astype(jnp.int32)
    out = _sorter(x, mag2, mask2)
    return out.reshape(B, N, D)
